# baseline (speedup 1.0000x reference)
import functools

import jax
import jax.numpy as jnp
import numpy as np
from jax import lax
from jax.experimental import pallas as pl
from jax.experimental.pallas import tpu as pltpu
from jax.experimental.pallas import tpu_sc as plsc

N = 10000
E = 160000
F = 256
R = 32
NC = 8
FC = 32
NTILES = 32
WIN = 40
NWIN = 128
EPT = WIN * NWIN
EPAD = NTILES * EPT
NPAD = 10240
NPS = NPAD // 16
ZROWS = 128


def _chunk_perm():
    p = np.arange(4 * F)
    c, r = p // 128, p % 128
    ch, k = r // FC, r % FC
    return jnp.asarray(ch * F + c * FC + k, dtype=jnp.int32)


def _phi_body(s_ref, v_ref, w1_ref, b1_ref, w2_ref, b2_ref, out_ref, vout_ref):
    x = s_ref[...]
    h = jnp.dot(x, w1_ref[...], preferred_element_type=jnp.float32) + b1_ref[...]
    h = h * jax.nn.sigmoid(h)
    ph = jnp.dot(h, w2_ref[...], preferred_element_type=jnp.float32) + b2_ref[...]
    z = jnp.zeros((v_ref.shape[0], FC), jnp.float32)
    for c in range(NC):
        out_ref[c] = ph[:, c * 128:(c + 1) * 128]
        vout_ref[c] = jnp.concatenate(
            [v_ref[:, m, c * FC:(c + 1) * FC] for m in range(3)] + [z], axis=1)


def _wmat_body(rad_ref, env_ref, wr_ref, br_ref, out_ref):
    w = jnp.dot(rad_ref[...], wr_ref[...], preferred_element_type=jnp.float32)
    w = (w + br_ref[...]) * env_ref[...]
    wu = lax.bitcast_convert_type(w, jnp.uint32)
    r = ((wu + jnp.uint32(0x7FFF) + ((wu >> 16) & jnp.uint32(1))) >> 16).astype(jnp.int32)
    for c in range(NC):
        blocks = []
        for ch in range(4):
            base = c * 128 + ch * 32
            lo = r[:, base:base + 16]
            hi = r[:, base + 16:base + 32]
            blocks.append(lo | (hi << 16))
        out_ref[c] = jnp.concatenate(blocks, axis=1)


def _combine_body(s_ref, v_ref, part_ref, so_ref, vo_ref):
    ps = part_ref[0] + part_ref[1]
    ds = jnp.concatenate([ps[c, :, 0:FC] for c in range(NC)], axis=-1)
    so_ref[:, 0, :] = s_ref[:, 0, :] + ds
    for m in range(3):
        dv = jnp.concatenate(
            [ps[c, :, (m + 1) * FC:(m + 2) * FC] for c in range(NC)], axis=-1)
        vo_ref[:, m, :] = v_ref[:, m, :] + dv


def _phi_chunks(s2, v, W1, b1, W2p, b2p):
    nb = 400
    return pl.pallas_call(
        _phi_body,
        grid=(N // nb,),
        in_specs=[
            pl.BlockSpec((nb, F), lambda i: (i, 0)),
            pl.BlockSpec((nb, 3, F), lambda i: (i, 0, 0)),
            pl.BlockSpec((F, F), lambda i: (0, 0)),
            pl.BlockSpec((F,), lambda i: (0,)),
            pl.BlockSpec((F, 4 * F), lambda i: (0, 0)),
            pl.BlockSpec((4 * F,), lambda i: (0,)),
        ],
        out_specs=[
            pl.BlockSpec((NC, nb, 128), lambda i: (0, i, 0)),
            pl.BlockSpec((NC, nb, 128), lambda i: (0, i, 0)),
        ],
        out_shape=[
            jax.ShapeDtypeStruct((NC, N, 128), jnp.float32),
            jax.ShapeDtypeStruct((NC, N, 128), jnp.float32),
        ],
    )(s2, v, W1, b1, W2p, b2p)


def _wmat_chunks(rad_p, env_p, Wrp, brp):
    eb = 800
    return pl.pallas_call(
        _wmat_body,
        grid=(E // eb,),
        in_specs=[
            pl.BlockSpec((eb, R), lambda i: (i, 0)),
            pl.BlockSpec((eb, 1), lambda i: (i, 0)),
            pl.BlockSpec((R, 4 * F), lambda i: (0, 0)),
            pl.BlockSpec((4 * F,), lambda i: (0,)),
        ],
        out_specs=pl.BlockSpec((NC, eb, 64), lambda i: (0, i, 0)),
        out_shape=jax.ShapeDtypeStruct((NC, EPAD, 64), jnp.int32),
    )(rad_p, env_p, Wrp, brp)


def _combine(s, v, partial):
    nb = 400
    return pl.pallas_call(
        _combine_body,
        grid=(N // nb,),
        in_specs=[
            pl.BlockSpec((nb, 1, F), lambda i: (i, 0, 0)),
            pl.BlockSpec((nb, 3, F), lambda i: (i, 0, 0)),
            pl.BlockSpec((2, NC, nb, 128), lambda i: (0, 0, i, 0)),
        ],
        out_specs=[
            pl.BlockSpec((nb, 1, F), lambda i: (i, 0, 0)),
            pl.BlockSpec((nb, 3, F), lambda i: (i, 0, 0)),
        ],
        out_shape=[
            jax.ShapeDtypeStruct((N, 1, F), jnp.float32),
            jax.ShapeDtypeStruct((N, 3, F), jnp.float32),
        ],
    )(s, v, partial)


def _sc_body(phi_hbm, v_hbm, w_hbm, uv_hbm, idxj_hbm, idxi_hbm, out_hbm,
             acc, idxj_v, idxi_v, uv_v, phi_b, v_b, w_b, upd_b,
             gsem0, gsem1, ssem0, ssem1, ssem2, ssem3):
    cid = lax.axis_index("c")
    sid = lax.axis_index("s")
    wid = sid * 2 + cid
    z16 = jnp.zeros((16,), jnp.float32)
    gsem = (gsem0, gsem1)
    ssem = (ssem0, ssem1, ssem2, ssem3)

    def _stage_issue(w, ib):
        pltpu.async_copy(idxj_hbm.at[wid, w], idxj_v.at[ib], ssem[ib])
        pltpu.async_copy(idxi_hbm.at[wid, w], idxi_v.at[ib], ssem[ib])
        pltpu.async_copy(uv_hbm.at[wid, w], uv_v.at[ib], ssem[ib])

    def _stage_wait(ib):
        pltpu.make_async_copy(idxj_hbm.at[wid, 0], idxj_v.at[ib], ssem[ib]).wait()
        pltpu.make_async_copy(idxi_hbm.at[wid, 0], idxi_v.at[ib], ssem[ib]).wait()
        pltpu.make_async_copy(uv_hbm.at[wid, 0], uv_v.at[ib], ssem[ib]).wait()

    def _gather_issue(c, w, ib, buf):
        e0 = wid * EPT + w * WIN
        pltpu.async_copy(phi_hbm.at[c].at[idxj_v.at[ib, 0]], phi_b.at[buf],
                         gsem[buf])
        pltpu.async_copy(v_hbm.at[c].at[idxj_v.at[ib, 0]], v_b.at[buf],
                         gsem[buf])
        pltpu.async_copy(w_hbm.at[c].at[pl.ds(e0, WIN)], w_b.at[buf], gsem[buf])

    def _gather_wait(c, buf):
        pltpu.make_async_copy(phi_hbm.at[c].at[pl.ds(0, WIN)], phi_b.at[buf],
                              gsem[buf]).wait()
        pltpu.make_async_copy(v_hbm.at[c].at[pl.ds(0, WIN)], v_b.at[buf],
                              gsem[buf]).wait()
        pltpu.make_async_copy(w_hbm.at[c].at[pl.ds(0, WIN)], w_b.at[buf],
                              gsem[buf]).wait()

    for c in range(NC):
        def _zrow(i, _):
            for kk in range(8):
                upd_b[i, pl.ds(kk * 16, 16)] = z16
            return 0

        lax.fori_loop(0, WIN, _zrow, 0)
        for i in range(NPS // WIN):
            pltpu.sync_copy(upd_b, acc.at[pl.ds(sid * NPS + i * WIN, WIN)])
        plsc.subcore_barrier()

        pltpu.sync_copy(idxj_hbm.at[wid, 0], idxj_v.at[0])
        pltpu.sync_copy(idxi_hbm.at[wid, 0], idxi_v.at[0])
        pltpu.sync_copy(uv_hbm.at[wid, 0], uv_v.at[0])
        _gather_issue(c, 0, 0, 0)
        _stage_issue(1, 1)

        def _quad(p, _):
            for sub in (0, 1, 2, 3):
                w = 4 * p + sub
                buf = sub % 2
                ib = sub

                _gather_wait(c, buf)

                @pl.when(w + 1 < NWIN)
                def _():
                    _stage_wait((ib + 1) % 4)
                    _gather_issue(c, w + 1, (ib + 1) % 4, 1 - buf)

                @pl.when(w + 2 < NWIN)
                def _():
                    _stage_issue(w + 2, (ib + 2) % 4)

                def _edge(e):
                    b0 = pl.multiple_of(e & ~15, 16)
                    lane = jnp.full((16,), e & 15, dtype=jnp.int32)
                    dnums = lax.GatherDimensionNumbers(
                        offset_dims=(), collapsed_slice_dims=(0,),
                        start_index_map=(0,))
                    ub = [lax.gather(uv_v[ib, m, pl.ds(b0, 16)],
                                     lane[:, None], dnums, slice_sizes=(1,),
                                     mode=lax.GatherScatterMode.PROMISE_IN_BOUNDS)
                          for m in range(3)]
                    Wp = []
                    for ch in range(4):
                        t = w_b[buf, e, pl.ds(ch * 16, 16)]
                        lo = lax.bitcast_convert_type(t << 16, jnp.float32)
                        hi = lax.bitcast_convert_type(
                            t & jnp.int32(-65536), jnp.float32)
                        Wp.append((lo, hi))
                    for k in (0, 1):
                        o = k * 16
                        ps = phi_b[buf, e, pl.ds(o, 16)]
                        pvv = phi_b[buf, e, pl.ds(32 + o, 16)]
                        pvs = phi_b[buf, e, pl.ds(64 + o, 16)]
                        pvc = phi_b[buf, e, pl.ds(96 + o, 16)]
                        vm = [v_b[buf, e, pl.ds(m * 32 + o, 16)]
                              for m in range(3)]
                        upd_b[e, pl.ds(o, 16)] = ps * Wp[0][k]
                        xvv = pvv * Wp[1][k]
                        xvs = pvs * Wp[2][k]
                        xvc = pvc * Wp[3][k]
                        for m in range(3):
                            ma, mb = (m + 1) % 3, (m + 2) % 3
                            cross = vm[ma] * ub[mb] - vm[mb] * ub[ma]
                            upd_b[e, pl.ds((m + 1) * 32 + o, 16)] = (
                                vm[m] * xvv + ub[m] * xvs + cross * xvc)

                plsc.parallel_loop(0, WIN, 1, unroll=2)(_edge)
                pltpu.sync_copy(upd_b, acc.at[idxi_v.at[ib, 0]], add=True)
            return 0

        lax.fori_loop(0, NWIN // 4, _quad, 0)
        plsc.subcore_barrier()
        pltpu.sync_copy(acc.at[pl.ds(sid * NPS, NPS)],
                        out_hbm.at[cid, c].at[pl.ds(sid * NPS, NPS)])


def _sc_middle(phi_r, v_r, w_r, uv_t, idxj_t, idxi_t):
    mesh = plsc.VectorSubcoreMesh(core_axis_name="c", subcore_axis_name="s")
    f = functools.partial(
        pl.kernel,
        out_type=jax.ShapeDtypeStruct((2, NC, NPAD, 128), jnp.float32),
        mesh=mesh,
        scratch_types=[
            pltpu.VMEM_SHARED((NPAD, 128), jnp.float32),
            pltpu.VMEM((4, 1, WIN), jnp.int32),
            pltpu.VMEM((4, 1, WIN), jnp.int32),
            pltpu.VMEM((4, 3, 48), jnp.float32),
            pltpu.VMEM((2, WIN, 128), jnp.float32),
            pltpu.VMEM((2, WIN, 128), jnp.float32),
            pltpu.VMEM((2, WIN, 64), jnp.int32),
            pltpu.VMEM((WIN, 128), jnp.float32),
            pltpu.SemaphoreType.DMA,
            pltpu.SemaphoreType.DMA,
            pltpu.SemaphoreType.DMA,
            pltpu.SemaphoreType.DMA,
            pltpu.SemaphoreType.DMA,
            pltpu.SemaphoreType.DMA,
        ],
    )(_sc_body)
    return f(phi_r, v_r, w_r, uv_t, idxj_t, idxi_t)


def kernel(s, v, radial_embeddings, envelope, unit_vectors, edge_index,
           W1, b1, W2, b2, Wr, br):
    perm = _chunk_perm()
    W2p, b2p = W2[:, perm], b2[perm]
    Wrp, brp = Wr[:, perm], br[perm]

    s2 = s[:, 0, :]
    phi_r, v_r = _phi_chunks(s2, v, W1, b1, W2p, b2p)

    rad = radial_embeddings[:, 0, :]
    pad_e = EPAD - E
    w_r = _wmat_chunks(rad, envelope, Wrp, brp)

    uv_p = jnp.pad(unit_vectors, ((0, pad_e), (0, 0)))
    pad_j = (jnp.arange(pad_e, dtype=jnp.int32) * 37) % N
    pad_i = N + ((jnp.arange(pad_e, dtype=jnp.int32) * 13) % (NPAD - N))
    idx_i = jnp.concatenate([edge_index[0], pad_i])
    idx_j = jnp.concatenate([edge_index[1], pad_j])

    uv_t = uv_p.reshape(NTILES, NWIN, WIN, 3).transpose(0, 1, 3, 2)
    uv_t = jnp.pad(uv_t, ((0, 0), (0, 0), (0, 0), (0, 48 - WIN)))
    idxj_t = idx_j.reshape(NTILES, NWIN, 1, WIN)
    idxi_t = idx_i.reshape(NTILES, NWIN, 1, WIN)

    partial = _sc_middle(phi_r, v_r, w_r, uv_t, idxj_t, idxi_t)

    return tuple(_combine(s, v, partial))


# speedup vs baseline: 1.2140x; 1.2140x over previous
import functools

import jax
import jax.numpy as jnp
import numpy as np
from jax import lax
from jax.experimental import pallas as pl
from jax.experimental.pallas import tpu as pltpu
from jax.experimental.pallas import tpu_sc as plsc

N = 10000
E = 160000
F = 256
R = 32
NC = 8
FC = 32
NTILES = 32
WIN = 40
NWIN = 128
EPT = WIN * NWIN
EPAD = NTILES * EPT
NPAD = 10240
NPS = NPAD // 16
ZROWS = 128


def _chunk_perm():
    p = np.arange(4 * F)
    c, r = p // 128, p % 128
    ch, k = r // FC, r % FC
    return jnp.asarray(ch * F + c * FC + k, dtype=jnp.int32)


def _phi_body(s_ref, v_ref, w1_ref, b1_ref, w2_ref, b2_ref, out_ref):
    x = s_ref[...]
    h = jnp.dot(x, w1_ref[...], preferred_element_type=jnp.float32) + b1_ref[...]
    h = h * jax.nn.sigmoid(h)
    ph = jnp.dot(h, w2_ref[...], preferred_element_type=jnp.float32) + b2_ref[...]
    z = jnp.zeros((v_ref.shape[0], FC), jnp.float32)
    for c in range(NC):
        out_ref[c] = jnp.concatenate(
            [ph[:, c * 128:(c + 1) * 128]]
            + [v_ref[:, m, c * FC:(c + 1) * FC] for m in range(3)] + [z],
            axis=1)


def _wmat_body(rad_ref, env_ref, wr_ref, br_ref, out_ref):
    w = jnp.dot(rad_ref[...], wr_ref[...], preferred_element_type=jnp.float32)
    w = (w + br_ref[...]) * env_ref[...]
    for c in range(NC):
        out_ref[c] = w[:, c * 128:(c + 1) * 128]


def _combine_body(s_ref, v_ref, part_ref, so_ref, vo_ref):
    ps = part_ref[0] + part_ref[1]
    ds = jnp.concatenate([ps[c, :, 0:FC] for c in range(NC)], axis=-1)
    so_ref[:, 0, :] = s_ref[:, 0, :] + ds
    for m in range(3):
        dv = jnp.concatenate(
            [ps[c, :, (m + 1) * FC:(m + 2) * FC] for c in range(NC)], axis=-1)
        vo_ref[:, m, :] = v_ref[:, m, :] + dv


def _phi_chunks(s2, v, W1, b1, W2p, b2p):
    nb = 400
    return pl.pallas_call(
        _phi_body,
        grid=(N // nb,),
        in_specs=[
            pl.BlockSpec((nb, F), lambda i: (i, 0)),
            pl.BlockSpec((nb, 3, F), lambda i: (i, 0, 0)),
            pl.BlockSpec((F, F), lambda i: (0, 0)),
            pl.BlockSpec((F,), lambda i: (0,)),
            pl.BlockSpec((F, 4 * F), lambda i: (0, 0)),
            pl.BlockSpec((4 * F,), lambda i: (0,)),
        ],
        out_specs=pl.BlockSpec((NC, nb, 256), lambda i: (0, i, 0)),
        out_shape=jax.ShapeDtypeStruct((NC, N, 256), jnp.float32),
    )(s2, v, W1, b1, W2p, b2p)


def _wmat_chunks(rad_p, env_p, Wrp, brp):
    eb = 800
    return pl.pallas_call(
        _wmat_body,
        grid=(E // eb,),
        in_specs=[
            pl.BlockSpec((eb, R), lambda i: (i, 0)),
            pl.BlockSpec((eb, 1), lambda i: (i, 0)),
            pl.BlockSpec((R, 4 * F), lambda i: (0, 0)),
            pl.BlockSpec((4 * F,), lambda i: (0,)),
        ],
        out_specs=pl.BlockSpec((NC, eb, 128), lambda i: (0, i, 0)),
        out_shape=jax.ShapeDtypeStruct((NC, EPAD, 128), jnp.float32),
    )(rad_p, env_p, Wrp, brp)


def _combine(s, v, partial):
    nb = 400
    return pl.pallas_call(
        _combine_body,
        grid=(N // nb,),
        in_specs=[
            pl.BlockSpec((nb, 1, F), lambda i: (i, 0, 0)),
            pl.BlockSpec((nb, 3, F), lambda i: (i, 0, 0)),
            pl.BlockSpec((2, NC, nb, 128), lambda i: (0, 0, i, 0)),
        ],
        out_specs=[
            pl.BlockSpec((nb, 1, F), lambda i: (i, 0, 0)),
            pl.BlockSpec((nb, 3, F), lambda i: (i, 0, 0)),
        ],
        out_shape=[
            jax.ShapeDtypeStruct((N, 1, F), jnp.float32),
            jax.ShapeDtypeStruct((N, 3, F), jnp.float32),
        ],
    )(s, v, partial)


def _sc_body(phi_hbm, w_hbm, uv_hbm, idxj_hbm, idxi_hbm, out_hbm,
             acc, idxj_v, idxi_v, uv_v, phi_b, w_b, upd_b,
             gsem0, gsem1, ssem0, ssem1, ssem2, ssem3):
    cid = lax.axis_index("c")
    sid = lax.axis_index("s")
    wid = sid * 2 + cid
    z16 = jnp.zeros((16,), jnp.float32)
    gsem = (gsem0, gsem1)
    ssem = (ssem0, ssem1, ssem2, ssem3)

    def _stage_issue(w, ib):
        pltpu.async_copy(idxj_hbm.at[wid, w], idxj_v.at[ib], ssem[ib])
        pltpu.async_copy(idxi_hbm.at[wid, w], idxi_v.at[ib], ssem[ib])
        pltpu.async_copy(uv_hbm.at[wid, w], uv_v.at[ib], ssem[ib])

    def _stage_wait(ib):
        pltpu.make_async_copy(idxj_hbm.at[wid, 0], idxj_v.at[ib], ssem[ib]).wait()
        pltpu.make_async_copy(idxi_hbm.at[wid, 0], idxi_v.at[ib], ssem[ib]).wait()
        pltpu.make_async_copy(uv_hbm.at[wid, 0], uv_v.at[ib], ssem[ib]).wait()

    def _gather_issue(c, w, ib, buf):
        e0 = wid * EPT + w * WIN
        pltpu.async_copy(phi_hbm.at[c].at[idxj_v.at[ib, 0]], phi_b.at[buf],
                         gsem[buf])
        pltpu.async_copy(w_hbm.at[c].at[pl.ds(e0, WIN)], w_b.at[buf], gsem[buf])

    def _gather_wait(c, buf):
        pltpu.make_async_copy(phi_hbm.at[c].at[pl.ds(0, WIN)], phi_b.at[buf],
                              gsem[buf]).wait()
        pltpu.make_async_copy(w_hbm.at[c].at[pl.ds(0, WIN)], w_b.at[buf],
                              gsem[buf]).wait()

    for c in range(NC):
        def _zrow(i, _):
            for kk in range(8):
                upd_b[i, pl.ds(kk * 16, 16)] = z16
            return 0

        lax.fori_loop(0, WIN, _zrow, 0)
        for i in range(NPS // WIN):
            pltpu.sync_copy(upd_b, acc.at[pl.ds(sid * NPS + i * WIN, WIN)])
        plsc.subcore_barrier()

        pltpu.sync_copy(idxj_hbm.at[wid, 0], idxj_v.at[0])
        pltpu.sync_copy(idxi_hbm.at[wid, 0], idxi_v.at[0])
        pltpu.sync_copy(uv_hbm.at[wid, 0], uv_v.at[0])
        _gather_issue(c, 0, 0, 0)
        _stage_issue(1, 1)

        def _quad(p, _):
            for sub in (0, 1, 2, 3):
                w = 4 * p + sub
                buf = sub % 2
                ib = sub

                _gather_wait(c, buf)

                @pl.when(w + 1 < NWIN)
                def _():
                    _stage_wait((ib + 1) % 4)
                    _gather_issue(c, w + 1, (ib + 1) % 4, 1 - buf)

                @pl.when(w + 2 < NWIN)
                def _():
                    _stage_issue(w + 2, (ib + 2) % 4)

                def _edge(e):
                    b0 = pl.multiple_of(e & ~15, 16)
                    lane = jnp.full((16,), e & 15, dtype=jnp.int32)
                    dnums = lax.GatherDimensionNumbers(
                        offset_dims=(), collapsed_slice_dims=(0,),
                        start_index_map=(0,))
                    ub = [lax.gather(uv_v[ib, m, pl.ds(b0, 16)],
                                     lane[:, None], dnums, slice_sizes=(1,),
                                     mode=lax.GatherScatterMode.PROMISE_IN_BOUNDS)
                          for m in range(3)]
                    for k in (0, 1):
                        o = k * 16
                        ps = phi_b[buf, e, pl.ds(o, 16)]
                        pvv = phi_b[buf, e, pl.ds(32 + o, 16)]
                        pvs = phi_b[buf, e, pl.ds(64 + o, 16)]
                        pvc = phi_b[buf, e, pl.ds(96 + o, 16)]
                        ws = w_b[buf, e, pl.ds(o, 16)]
                        wvv = w_b[buf, e, pl.ds(32 + o, 16)]
                        wvs = w_b[buf, e, pl.ds(64 + o, 16)]
                        wvc = w_b[buf, e, pl.ds(96 + o, 16)]
                        vm = [phi_b[buf, e, pl.ds(128 + m * 32 + o, 16)]
                              for m in range(3)]
                        upd_b[e, pl.ds(o, 16)] = ps * ws
                        xvv = pvv * wvv
                        xvs = pvs * wvs
                        xvc = pvc * wvc
                        for m in range(3):
                            ma, mb = (m + 1) % 3, (m + 2) % 3
                            cross = vm[ma] * ub[mb] - vm[mb] * ub[ma]
                            upd_b[e, pl.ds((m + 1) * 32 + o, 16)] = (
                                vm[m] * xvv + ub[m] * xvs + cross * xvc)

                plsc.parallel_loop(0, WIN, 1, unroll=1)(_edge)
                pltpu.sync_copy(upd_b, acc.at[idxi_v.at[ib, 0]], add=True)
            return 0

        lax.fori_loop(0, NWIN // 4, _quad, 0)
        plsc.subcore_barrier()
        pltpu.sync_copy(acc.at[pl.ds(sid * NPS, NPS)],
                        out_hbm.at[cid, c].at[pl.ds(sid * NPS, NPS)])


def _sc_middle(phi_r, w_r, uv_t, idxj_t, idxi_t):
    mesh = plsc.VectorSubcoreMesh(core_axis_name="c", subcore_axis_name="s")
    f = functools.partial(
        pl.kernel,
        out_type=jax.ShapeDtypeStruct((2, NC, NPAD, 128), jnp.float32),
        mesh=mesh,
        scratch_types=[
            pltpu.VMEM_SHARED((NPAD, 128), jnp.float32),
            pltpu.VMEM((4, 1, WIN), jnp.int32),
            pltpu.VMEM((4, 1, WIN), jnp.int32),
            pltpu.VMEM((4, 3, 48), jnp.float32),
            pltpu.VMEM((2, WIN, 256), jnp.float32),
            pltpu.VMEM((2, WIN, 128), jnp.float32),
            pltpu.VMEM((WIN, 128), jnp.float32),
            pltpu.SemaphoreType.DMA,
            pltpu.SemaphoreType.DMA,
            pltpu.SemaphoreType.DMA,
            pltpu.SemaphoreType.DMA,
            pltpu.SemaphoreType.DMA,
            pltpu.SemaphoreType.DMA,
        ],
    )(_sc_body)
    return f(phi_r, w_r, uv_t, idxj_t, idxi_t)


def kernel(s, v, radial_embeddings, envelope, unit_vectors, edge_index,
           W1, b1, W2, b2, Wr, br):
    perm = _chunk_perm()
    W2p, b2p = W2[:, perm], b2[perm]
    Wrp, brp = Wr[:, perm], br[perm]

    s2 = s[:, 0, :]
    phi_r = _phi_chunks(s2, v, W1, b1, W2p, b2p)

    rad = radial_embeddings[:, 0, :]
    pad_e = EPAD - E
    w_r = _wmat_chunks(rad, envelope, Wrp, brp)

    uv_p = jnp.pad(unit_vectors, ((0, pad_e), (0, 0)))
    pad_j = (jnp.arange(pad_e, dtype=jnp.int32) * 37) % N
    pad_i = N + ((jnp.arange(pad_e, dtype=jnp.int32) * 13) % (NPAD - N))
    idx_i = jnp.concatenate([edge_index[0], pad_i])
    idx_j = jnp.concatenate([edge_index[1], pad_j])

    uv_t = uv_p.reshape(NTILES, NWIN, WIN, 3).transpose(0, 1, 3, 2)
    uv_t = jnp.pad(uv_t, ((0, 0), (0, 0), (0, 0), (0, 48 - WIN)))
    idxj_t = idx_j.reshape(NTILES, NWIN, 1, WIN)
    idxi_t = idx_i.reshape(NTILES, NWIN, 1, WIN)

    partial = _sc_middle(phi_r, w_r, uv_t, idxj_t, idxi_t)

    return tuple(_combine(s, v, partial))
